# baseline (speedup 1.0000x reference)
import math

import jax
import jax.numpy as jnp
from jax.experimental import pallas as pl

_F32 = jnp.float32
_I32 = jnp.int32
_DEG2RAD = math.pi / 180.0


def _gwd_kernel(pred_ab_ref, pred_ang_ref, pred_hm_ref, target_hm_ref,
                misc_ref, out_ref):
    b = pl.program_id(0)

    featA = pred_ab_ref[0, 0]
    featB = pred_ab_ref[0, 1]
    featG = pred_ang_ref[0, 0]
    Vp0 = pred_hm_ref[0, 0]
    Vt0 = target_hm_ref[0, 0]
    m = misc_ref[0]

    sub_i = jax.lax.broadcasted_iota(_I32, (128, 128), 0)
    lan_i = jax.lax.broadcasted_iota(_I32, (128, 128), 1)
    flat_i = sub_i * 128 + lan_i
    klane = jax.lax.broadcasted_iota(_I32, (1, 128), 1)

    def extract(k, V, ys, xs):
        mx = jnp.max(V, keepdims=True)
        big = jnp.where(V == mx, flat_i, 99999)
        idx = jnp.min(big, keepdims=True)
        V = jnp.where(big == idx, -1.0, V)
        y = (idx // 128).astype(_F32)
        x = (idx % 128).astype(_F32)
        oh = klane == k
        ys = ys + jnp.where(oh, y, 0.0)
        xs = xs + jnp.where(oh, x, 0.0)
        return V, ys, xs

    z = jnp.zeros((1, 128), _F32)

    def body(k, carry):
        Vp, Vt, ysp, xsp, yst, xst = carry
        Vp, ysp, xsp = extract(k, Vp, ysp, xsp)
        Vt, yst, xst = extract(k, Vt, yst, xst)
        return (Vp, Vt, ysp, xsp, yst, xst)

    _, _, ysp, xsp, yst, xst = jax.lax.fori_loop(
        0, 100, body, (Vp0, Vt0, z, z, z, z))

    eye = (jax.lax.broadcasted_iota(_I32, (128, 128), 0) ==
           jax.lax.broadcasted_iota(_I32, (128, 128), 1)).astype(_F32)
    stacked = jnp.concatenate([ysp, xsp, yst, xst], axis=0)
    cols = jax.lax.dot_general(eye, stacked, (((1,), (1,)), ((), ())),
                               preferred_element_type=_F32)
    ys_p = cols[:, 0:1]
    xs_p = cols[:, 1:2]
    ys_t = cols[:, 2:3]
    xs_t = cols[:, 3:4]

    ta = m[:, 0:1]
    tb = m[:, 1:2]
    tang = m[:, 2:3]
    mask = m[:, 3:4]
    row = m[:, 4:5]
    col = m[:, 5:6]

    li = jax.lax.broadcasted_iota(_I32, (1, 128), 1).astype(_F32)
    ohr = (li == row).astype(_F32)
    ohc = (li == col).astype(_F32)

    featcat = jnp.concatenate([featA, featB, featG], axis=1)
    rows_sel = jnp.dot(ohr, featcat, preferred_element_type=_F32)
    a_g = jnp.sum(rows_sel[:, 0:128] * ohc, axis=1, keepdims=True)
    b_g = jnp.sum(rows_sel[:, 128:256] * ohc, axis=1, keepdims=True)
    g_g = jnp.sum(rows_sel[:, 256:384] * ohc, axis=1, keepdims=True)

    yp = ys_p * mask
    xp = xs_p * mask
    wp = a_g * 2.0 * mask
    hp = b_g * 2.0 * mask
    angp = (g_g - 90.0) * mask

    yt = ys_t * mask
    xt = xs_t * mask
    wt = ta * 2.0 * mask
    ht = tb * 2.0 * mask
    angt = (tang - 90.0) * mask

    xy_dist = jnp.square(yp - yt) + jnp.square(xp - xt)

    wp_ = jnp.clip(wp, 1e-07, 10000000.0)
    hp_ = jnp.clip(hp, 1e-07, 10000000.0)
    wt_ = jnp.clip(wt, 1e-07, 10000000.0)
    ht_ = jnp.clip(ht, 1e-07, 10000000.0)

    rp = angp * _DEG2RAD
    rt = angt * _DEG2RAD
    cp = jnp.cos(rp)
    sp = jnp.sin(rp)
    ct = jnp.cos(rt)
    st = jnp.sin(rt)

    dap = 0.5 * wp_
    dbp = 0.5 * hp_
    dat = 0.5 * wt_
    dbt = 0.5 * ht_
    a2p = dap * dap
    b2p = dbp * dbp
    a2t = dat * dat
    b2t = dbt * dbt

    Sp11 = cp * cp * a2p + sp * sp * b2p
    Sp22 = sp * sp * a2p + cp * cp * b2p
    Sp12 = cp * sp * (a2p - b2p)
    St11 = ct * ct * a2t + st * st * b2t
    St22 = st * st * a2t + ct * ct * b2t
    St12 = ct * st * (a2t - b2t)

    tr = Sp11 * St11 + 2.0 * Sp12 * St12 + Sp22 * St22
    det_sqrt = (dap * dbp) * (dat * dbt)
    whr = (a2p + b2p) + (a2t + b2t)
    whr = whr - 2.0 * jnp.sqrt(jnp.clip(tr + 2.0 * det_sqrt, 0.0, None))

    distance = jnp.clip(xy_dist + whr, 0.0, None)
    distance = jnp.log1p(distance)
    lossv = 1.0 - 1.0 / (1.0 + distance)

    lsum = jnp.sum(lossv)
    msum = jnp.sum(mask)

    @pl.when(b == 0)
    def _():
        out_ref[...] = jnp.zeros((1, 128), _F32)

    out_ref[...] += (jnp.where(klane == 0, lsum, 0.0) +
                     jnp.where(klane == 1, msum, 0.0))


def kernel(pred_ab, pred_ang, pred_hm, target_ab, target_ang, target_hm, ind, reg_mask):
    B, C, H, W = pred_ab.shape
    K = ind.shape[1]

    mask = reg_mask.astype(_F32)
    ind32 = ind.astype(_I32)
    row_f = (ind32 // W).astype(_F32)
    col_f = (ind32 % W).astype(_F32)

    fields = [
        target_ab[:, :, 0], target_ab[:, :, 1], target_ang[:, :, 0],
        mask, row_f, col_f,
    ]
    misc = jnp.stack(fields, axis=2)
    misc = jnp.pad(misc, ((0, 0), (0, 128 - K), (0, 16 - misc.shape[2])))

    out = pl.pallas_call(
        _gwd_kernel,
        grid=(B,),
        in_specs=[
            pl.BlockSpec((1, C, H, W), lambda b: (b, 0, 0, 0)),
            pl.BlockSpec((1, 1, H, W), lambda b: (b, 0, 0, 0)),
            pl.BlockSpec((1, 1, H, W), lambda b: (b, 0, 0, 0)),
            pl.BlockSpec((1, 1, H, W), lambda b: (b, 0, 0, 0)),
            pl.BlockSpec((1, 128, 16), lambda b: (b, 0, 0)),
        ],
        out_specs=pl.BlockSpec((1, 128), lambda b: (0, 0)),
        out_shape=jax.ShapeDtypeStruct((1, 128), _F32),
    )(pred_ab, pred_ang, pred_hm, target_hm, misc)

    return out[0, 0] / (out[0, 1] + 1e-08)
